# Initial kernel scaffold; baseline (speedup 1.0000x reference)
import jax
import jax.numpy as jnp
from jax.experimental import pallas as pl


def kernel(h, g, pos, vel, edge_index, params):
    raise NotImplementedError("write your pallas kernel here")


# speedup vs baseline: 3.3180x; 3.3180x over previous
import functools

import jax
import jax.numpy as jnp
from jax import lax
from jax.experimental import pallas as pl
from jax.experimental.pallas import tpu as pltpu
from jax.experimental.pallas import tpu_sc as plsc

DT = 0.01
BOX = 10.0

NC = 2
NS = 16
NW = NC * NS
DTBL = 80
SUB = 128
CHUNK = 512
NSUB = CHUNK // SUB


def _silu(x):
    return x * jax.nn.sigmoid(x)


def _make_gather(e_pad):
    per_tile = e_pad // NW
    n_chunks = per_tile // CHUNK
    rows_per_tile = per_tile // SUB
    mesh = plsc.VectorSubcoreMesh(
        core_axis_name="c", subcore_axis_name="s", num_cores=NC, num_subcores=NS)

    @functools.partial(
        pl.kernel, mesh=mesh,
        out_type=(jax.ShapeDtypeStruct((e_pad, DTBL), jnp.float32),
                  jax.ShapeDtypeStruct((e_pad, DTBL), jnp.float32)),
        scratch_types=[
            pltpu.VMEM((NSUB, SUB), jnp.int32),
            pltpu.VMEM((NSUB, SUB), jnp.int32),
            pltpu.VMEM((CHUNK, DTBL), jnp.float32),
            pltpu.VMEM((CHUNK, DTBL), jnp.float32),
            pltpu.SemaphoreType.DMA,
        ],
        compiler_params=pltpu.CompilerParams(use_tc_tiling_on_sc=False),
    )
    def gather_k(row_hbm, col_hbm, tr_hbm, tc_hbm, gr_hbm, gc_hbm,
                 idx_r, idx_c, grbuf, gcbuf, sem):
        wid = lax.axis_index("s") * NC + lax.axis_index("c")

        def chunk(k, carry):
            rbase = wid * rows_per_tile + k * NSUB
            pltpu.sync_copy(row_hbm.at[pl.ds(rbase, NSUB)], idx_r)
            pltpu.sync_copy(col_hbm.at[pl.ds(rbase, NSUB)], idx_c)
            cps = []
            for j in range(NSUB):
                cps.append(pltpu.async_copy(
                    tr_hbm.at[idx_r.at[j]], grbuf.at[pl.ds(j * SUB, SUB)], sem))
                cps.append(pltpu.async_copy(
                    tc_hbm.at[idx_c.at[j]], gcbuf.at[pl.ds(j * SUB, SUB)], sem))
            for cp in cps:
                cp.wait()
            ebase = wid * per_tile + k * CHUNK
            pltpu.sync_copy(grbuf, gr_hbm.at[pl.ds(ebase, CHUNK)])
            pltpu.sync_copy(gcbuf, gc_hbm.at[pl.ds(ebase, CHUNK)])
            return carry

        lax.fori_loop(0, n_chunks, chunk, 0)

    return gather_k


def _make_scatter(e_pad, n_nodes):
    per_tile = e_pad // NW
    n_chunks = per_tile // CHUNK
    rows_per_tile = per_tile // SUB
    rows_per_sub = n_nodes // NS
    mesh = plsc.VectorSubcoreMesh(
        core_axis_name="c", subcore_axis_name="s", num_cores=NC, num_subcores=NS)

    @functools.partial(
        pl.kernel, mesh=mesh,
        out_type=jax.ShapeDtypeStruct((NC, n_nodes, DTBL), jnp.float32),
        scratch_types=[
            pltpu.VMEM_SHARED((n_nodes, DTBL), jnp.float32),
            pltpu.VMEM((NSUB, SUB), jnp.int32),
            pltpu.VMEM((CHUNK, DTBL), jnp.float32),
        ],
        compiler_params=pltpu.CompilerParams(use_tc_tiling_on_sc=False),
    )
    def scatter_k(row_hbm, vals_hbm, zeros_hbm, out_hbm, acc, idx_r, vbuf):
        cid = lax.axis_index("c")
        sid = lax.axis_index("s")
        wid = sid * NC + cid
        nbase = sid * rows_per_sub
        pltpu.sync_copy(zeros_hbm.at[pl.ds(nbase, rows_per_sub)],
                        acc.at[pl.ds(nbase, rows_per_sub)])
        plsc.subcore_barrier()

        def chunk(k, carry):
            rbase = wid * rows_per_tile + k * NSUB
            pltpu.sync_copy(row_hbm.at[pl.ds(rbase, NSUB)], idx_r)
            ebase = wid * per_tile + k * CHUNK
            pltpu.sync_copy(vals_hbm.at[pl.ds(ebase, CHUNK)], vbuf)
            for j in range(NSUB):
                pltpu.sync_copy(vbuf.at[pl.ds(j * SUB, SUB)],
                                acc.at[idx_r.at[j]], add=True)
            return carry

        lax.fori_loop(0, n_chunks, chunk, 0)
        plsc.subcore_barrier()
        pltpu.sync_copy(acc.at[pl.ds(nbase, rows_per_sub)],
                        out_hbm.at[cid, pl.ds(nbase, rows_per_sub)])

    return scatter_k


def _table_body(h_ref, pos4_ref, w1r_ref, w1c_ref, tr_ref, tc_ref):
    h = h_ref[...]
    p4 = pos4_ref[...]
    pad = jnp.zeros((h.shape[0], DTBL - 68), jnp.float32)
    hr = jnp.dot(h, w1r_ref[...], preferred_element_type=jnp.float32)
    hc = jnp.dot(h, w1c_ref[...], preferred_element_type=jnp.float32)
    tr_ref[...] = jnp.concatenate([hr, p4, pad], axis=-1)
    tc_ref[...] = jnp.concatenate([hc, p4, pad], axis=-1)


def _edge_body(n_real, blk, gr_ref, gc_ref, w1d_ref, be1_ref, we2_ref, be2_ref,
               wc1_ref, bc1_ref, wc2_ref, bc2_ref, out_ref):
    gr = gr_ref[...]
    gc = gc_ref[...]
    diff = gr[:, 64:68] - gc[:, 64:68]
    dist2 = jnp.sum(diff * diff, axis=-1, keepdims=True)
    pre = gr[:, :64] + gc[:, :64] + dist2 * w1d_ref[...] + be1_ref[...]
    a1 = _silu(pre)
    m = _silu(jnp.dot(a1, we2_ref[...], preferred_element_type=jnp.float32)
              + be2_ref[...])
    t = _silu(jnp.dot(m, wc1_ref[...], preferred_element_type=jnp.float32)
              + bc1_ref[...])
    c = jnp.sum(t * wc2_ref[...], axis=-1, keepdims=True) + bc2_ref[0, 0]
    eid = pl.program_id(0) * blk + lax.broadcasted_iota(jnp.int32, (blk, 1), 0)
    live = jnp.where(eid < n_real, 1.0, 0.0).astype(jnp.float32)
    m = m * live
    fc = diff * (c * live)
    pad = jnp.zeros((blk, DTBL - 68), jnp.float32)
    out_ref[...] = jnp.concatenate([m, fc, pad], axis=-1)


def _node_body(blk, h_ref, g_ref, pos4_ref, vel4_ref, p0_ref, p1_ref,
               wn1h_ref, wn1a_ref, bn1_ref, wn2_ref, bn2_ref,
               wqh_ref, wqa_ref, bq_ref, nw1r_ref, nw1c_ref,
               h_out, g_out, pos4_out, vel4_out, tr_out, tc_out, qs_out):
    agg = p0_ref[0] + p1_ref[0]
    aggm = agg[:, :64]
    f4 = agg[:, 64:68]
    h = h_ref[...]
    pre1 = (jnp.dot(h, wn1h_ref[...], preferred_element_type=jnp.float32)
            + jnp.dot(aggm, wn1a_ref[...], preferred_element_type=jnp.float32)
            + bn1_ref[...])
    big_g = (jnp.dot(_silu(pre1), wn2_ref[...],
                     preferred_element_type=jnp.float32) + bn2_ref[...])
    q = (jnp.sum(h * wqh_ref[...], axis=-1, keepdims=True)
         + jnp.sum(aggm * wqa_ref[...], axis=-1, keepdims=True)
         + bq_ref[0, 0])
    q = jnp.tanh(q)
    vel4 = jnp.exp(q) * vel4_ref[...] + f4 * DT
    g_new = g_ref[...] + big_g * DT
    pos4 = jnp.mod(pos4_ref[...] + vel4 * DT, BOX)
    h_new = h + g_new * DT
    h_out[...] = h_new
    g_out[...] = g_new
    pos4_out[...] = pos4
    vel4_out[...] = vel4
    pad = jnp.zeros((blk, DTBL - 68), jnp.float32)
    nr = jnp.dot(h_new, nw1r_ref[...], preferred_element_type=jnp.float32)
    nc2 = jnp.dot(h_new, nw1c_ref[...], preferred_element_type=jnp.float32)
    tr_out[...] = jnp.concatenate([nr, pos4, pad], axis=-1)
    tc_out[...] = jnp.concatenate([nc2, pos4, pad], axis=-1)

    @pl.when(pl.program_id(0) == 0)
    def _():
        qs_out[...] = jnp.zeros((1, 1), jnp.float32)
    qs_out[...] += jnp.sum(q).reshape(1, 1)


def _full_spec(shape):
    return pl.BlockSpec(shape, lambda i: tuple(0 for _ in shape))


def kernel(h, g, pos, vel, edge_index, params):
    n = h.shape[0]
    e = edge_index.shape[1]
    e_pad = ((e + NW * CHUNK - 1) // (NW * CHUNK)) * (NW * CHUNK)

    row = edge_index[0]
    col = edge_index[1]
    pad_e = e_pad - e
    row2d = jnp.concatenate([row, jnp.zeros((pad_e,), jnp.int32)]).reshape(
        e_pad // SUB, SUB)
    col2d = jnp.concatenate([col, jnp.zeros((pad_e,), jnp.int32)]).reshape(
        e_pad // SUB, SUB)
    pos4 = jnp.concatenate([pos, jnp.zeros((n, 1), jnp.float32)], axis=-1)
    vel4 = jnp.concatenate([vel, jnp.zeros((n, 1), jnp.float32)], axis=-1)
    zeros_tbl = jnp.zeros((n, DTBL), jnp.float32)

    gather_k = _make_gather(e_pad)
    scatter_k = _make_scatter(e_pad, n)

    bn = 2000
    n_blocks = n // bn

    def table_call(hh, pp4, w1r, w1c):
        return pl.pallas_call(
            _table_body,
            grid=(n_blocks,),
            in_specs=[
                pl.BlockSpec((bn, 128), lambda i: (i, 0)),
                pl.BlockSpec((bn, 4), lambda i: (i, 0)),
                _full_spec((128, 64)),
                _full_spec((128, 64)),
            ],
            out_specs=[pl.BlockSpec((bn, DTBL), lambda i: (i, 0))] * 2,
            out_shape=[jax.ShapeDtypeStruct((n, DTBL), jnp.float32)] * 2,
        )(hh, pp4, w1r, w1c)

    be = 4096
    e_blocks = e_pad // be

    def edge_call(gr, gc, p):
        w1d = p['We1'][256:257, :]
        be1 = p['be1'].reshape(1, 64)
        be2 = p['be2'].reshape(1, 64)
        bc1 = p['bc1'].reshape(1, 64)
        wc2 = p['Wc2'].reshape(1, 64)
        bc2 = p['bc2'].reshape(1, 1)
        return pl.pallas_call(
            functools.partial(_edge_body, e, be),
            grid=(e_blocks,),
            in_specs=[
                pl.BlockSpec((be, DTBL), lambda i: (i, 0)),
                pl.BlockSpec((be, DTBL), lambda i: (i, 0)),
                _full_spec((1, 64)), _full_spec((1, 64)),
                _full_spec((64, 64)), _full_spec((1, 64)),
                _full_spec((64, 64)), _full_spec((1, 64)),
                _full_spec((1, 64)), _full_spec((1, 1)),
            ],
            out_specs=pl.BlockSpec((be, DTBL), lambda i: (i, 0)),
            out_shape=jax.ShapeDtypeStruct((e_pad, DTBL), jnp.float32),
        )(gr, gc, w1d, be1, p['We2'], be2, p['Wc1'], bc1, wc2, bc2)

    def node_call(hh, gg, pp4, vv4, parts, p, p_next):
        wn1h = p['Wn1'][:128, :]
        wn1a = p['Wn1'][128:, :]
        wqh = p['Wq'][:128, 0].reshape(1, 128)
        wqa = p['Wq'][128:, 0].reshape(1, 64)
        return pl.pallas_call(
            functools.partial(_node_body, bn),
            grid=(n_blocks,),
            in_specs=[
                pl.BlockSpec((bn, 128), lambda i: (i, 0)),
                pl.BlockSpec((bn, 128), lambda i: (i, 0)),
                pl.BlockSpec((bn, 4), lambda i: (i, 0)),
                pl.BlockSpec((bn, 4), lambda i: (i, 0)),
                pl.BlockSpec((1, bn, DTBL), lambda i: (0, i, 0)),
                pl.BlockSpec((1, bn, DTBL), lambda i: (1, i, 0)),
                _full_spec((128, 64)), _full_spec((64, 64)),
                _full_spec((1, 64)),
                _full_spec((64, 128)), _full_spec((1, 128)),
                _full_spec((1, 128)), _full_spec((1, 64)),
                _full_spec((1, 1)),
                _full_spec((128, 64)), _full_spec((128, 64)),
            ],
            out_specs=[
                pl.BlockSpec((bn, 128), lambda i: (i, 0)),
                pl.BlockSpec((bn, 128), lambda i: (i, 0)),
                pl.BlockSpec((bn, 4), lambda i: (i, 0)),
                pl.BlockSpec((bn, 4), lambda i: (i, 0)),
                pl.BlockSpec((bn, DTBL), lambda i: (i, 0)),
                pl.BlockSpec((bn, DTBL), lambda i: (i, 0)),
                pl.BlockSpec((1, 1), lambda i: (0, 0)),
            ],
            out_shape=[
                jax.ShapeDtypeStruct((n, 128), jnp.float32),
                jax.ShapeDtypeStruct((n, 128), jnp.float32),
                jax.ShapeDtypeStruct((n, 4), jnp.float32),
                jax.ShapeDtypeStruct((n, 4), jnp.float32),
                jax.ShapeDtypeStruct((n, DTBL), jnp.float32),
                jax.ShapeDtypeStruct((n, DTBL), jnp.float32),
                jax.ShapeDtypeStruct((1, 1), jnp.float32),
            ],
        )(hh, gg, pp4, vv4, parts, parts, wn1h, wn1a, p['bn1'].reshape(1, 64),
          p['Wn2'], p['bn2'].reshape(1, 128), wqh, wqa,
          p['bq'].reshape(1, 1), p_next['We1'][:128, :],
          p_next['We1'][128:256, :])

    p0 = params[0]
    tr, tc = table_call(h, pos4, p0['We1'][:128, :], p0['We1'][128:256, :])
    ldj = jnp.zeros((), jnp.float32)
    for it, p in enumerate(params):
        gr, gc = gather_k(row2d, col2d, tr, tc)
        vals = edge_call(gr, gc, p)
        parts = scatter_k(row2d, vals, zeros_tbl)
        p_next = params[it + 1] if it + 1 < len(params) else params[0]
        h, g, pos4, vel4, tr, tc, qs = node_call(
            h, g, pos4, vel4, parts, p, p_next)
        ldj = ldj + qs[0, 0]

    return h, g, pos4[:, :3], vel4[:, :3], ldj


# speedup vs baseline: 3.5334x; 1.0649x over previous
import functools

import jax
import jax.numpy as jnp
from jax import lax
from jax.experimental import pallas as pl
from jax.experimental.pallas import tpu as pltpu
from jax.experimental.pallas import tpu_sc as plsc

DT = 0.01
BOX = 10.0

NC = 2
NS = 16
NW = NC * NS
DTBL = 80
SUB = 128
CHUNK = 512
NSUB = CHUNK // SUB


def _silu(x):
    return x * jax.nn.sigmoid(x)


def _make_gather(e_pad):
    per_tile = e_pad // NW
    chunk = 256
    nsub = chunk // SUB
    n_chunks = per_tile // chunk
    n_pairs = n_chunks // 2
    rows_per_tile = per_tile // SUB
    mesh = plsc.VectorSubcoreMesh(
        core_axis_name="c", subcore_axis_name="s", num_cores=NC, num_subcores=NS)

    @functools.partial(
        pl.kernel, mesh=mesh,
        out_type=(jax.ShapeDtypeStruct((e_pad, DTBL), jnp.float32),
                  jax.ShapeDtypeStruct((e_pad, DTBL), jnp.float32)),
        scratch_types=[
            pltpu.VMEM((2, nsub, SUB), jnp.int32),
            pltpu.VMEM((2, nsub, SUB), jnp.int32),
            pltpu.VMEM((2, chunk, DTBL), jnp.float32),
            pltpu.VMEM((2, chunk, DTBL), jnp.float32),
            [pltpu.SemaphoreType.DMA] * 2,
            [pltpu.SemaphoreType.DMA] * 2,
            [pltpu.SemaphoreType.DMA] * 2,
        ],
        compiler_params=pltpu.CompilerParams(use_tc_tiling_on_sc=False),
    )
    def gather_k(row_hbm, col_hbm, tr_hbm, tc_hbm, gr_hbm, gc_hbm,
                 idx_r, idx_c, grbuf, gcbuf, semg, semo, semi):
        wid = lax.axis_index("s") * NC + lax.axis_index("c")
        row0 = wid * rows_per_tile
        e0 = wid * per_tile

        def fetch_idx(k, b):
            rbase = row0 + k * nsub
            pltpu.async_copy(row_hbm.at[pl.ds(rbase, nsub)], idx_r.at[b],
                             semi[b])
            pltpu.async_copy(col_hbm.at[pl.ds(rbase, nsub)], idx_c.at[b],
                             semi[b])

        def wait_idx(b):
            pltpu.make_async_copy(row_hbm.at[pl.ds(0, nsub)], idx_r.at[b],
                                  semi[b]).wait()
            pltpu.make_async_copy(col_hbm.at[pl.ds(0, nsub)], idx_c.at[b],
                                  semi[b]).wait()

        def issue_gathers(b):
            for j in range(nsub):
                pltpu.async_copy(tr_hbm.at[idx_r.at[b, j]],
                                 grbuf.at[b, pl.ds(j * SUB, SUB)], semg[b])
                pltpu.async_copy(tc_hbm.at[idx_c.at[b, j]],
                                 gcbuf.at[b, pl.ds(j * SUB, SUB)], semg[b])

        def wait_gathers(b):
            for j in range(nsub):
                pltpu.make_async_copy(tr_hbm.at[idx_r.at[b, j]],
                                      grbuf.at[b, pl.ds(j * SUB, SUB)],
                                      semg[b]).wait()
                pltpu.make_async_copy(tc_hbm.at[idx_c.at[b, j]],
                                      gcbuf.at[b, pl.ds(j * SUB, SUB)],
                                      semg[b]).wait()

        def issue_outs(k, b):
            ebase = e0 + k * chunk
            pltpu.async_copy(grbuf.at[b], gr_hbm.at[pl.ds(ebase, chunk)],
                             semo[b])
            pltpu.async_copy(gcbuf.at[b], gc_hbm.at[pl.ds(ebase, chunk)],
                             semo[b])

        def wait_outs(b):
            pltpu.make_async_copy(grbuf.at[b], gr_hbm.at[pl.ds(0, chunk)],
                                  semo[b]).wait()
            pltpu.make_async_copy(gcbuf.at[b], gc_hbm.at[pl.ds(0, chunk)],
                                  semo[b]).wait()

        for b in (0, 1):
            fetch_idx(b, b)
            wait_idx(b)
            issue_gathers(b)

        def body(i, carry):
            for b in (0, 1):
                k = 2 * i + b
                wait_gathers(b)
                fetch_idx(k + 2, b)
                issue_outs(k, b)
                wait_outs(b)
                wait_idx(b)
                issue_gathers(b)
            return carry

        lax.fori_loop(0, n_pairs - 1, body, 0)
        for b in (0, 1):
            k = 2 * (n_pairs - 1) + b
            wait_gathers(b)
            issue_outs(k, b)
            wait_outs(b)

    return gather_k


def _make_scatter(e_pad, n_nodes):
    per_tile = e_pad // NW
    n_chunks = per_tile // CHUNK
    rows_per_tile = per_tile // SUB
    rows_per_sub = n_nodes // NS
    mesh = plsc.VectorSubcoreMesh(
        core_axis_name="c", subcore_axis_name="s", num_cores=NC, num_subcores=NS)

    @functools.partial(
        pl.kernel, mesh=mesh,
        out_type=jax.ShapeDtypeStruct((NC, n_nodes, DTBL), jnp.float32),
        scratch_types=[
            pltpu.VMEM_SHARED((n_nodes, DTBL), jnp.float32),
            pltpu.VMEM((NSUB, SUB), jnp.int32),
            pltpu.VMEM((CHUNK, DTBL), jnp.float32),
        ],
        compiler_params=pltpu.CompilerParams(use_tc_tiling_on_sc=False),
    )
    def scatter_k(row_hbm, vals_hbm, zeros_hbm, out_hbm, acc, idx_r, vbuf):
        cid = lax.axis_index("c")
        sid = lax.axis_index("s")
        wid = sid * NC + cid
        nbase = sid * rows_per_sub
        pltpu.sync_copy(zeros_hbm.at[pl.ds(nbase, rows_per_sub)],
                        acc.at[pl.ds(nbase, rows_per_sub)])
        plsc.subcore_barrier()

        def chunk(k, carry):
            rbase = wid * rows_per_tile + k * NSUB
            pltpu.sync_copy(row_hbm.at[pl.ds(rbase, NSUB)], idx_r)
            ebase = wid * per_tile + k * CHUNK
            pltpu.sync_copy(vals_hbm.at[pl.ds(ebase, CHUNK)], vbuf)
            for j in range(NSUB):
                pltpu.sync_copy(vbuf.at[pl.ds(j * SUB, SUB)],
                                acc.at[idx_r.at[j]], add=True)
            return carry

        lax.fori_loop(0, n_chunks, chunk, 0)
        plsc.subcore_barrier()
        pltpu.sync_copy(acc.at[pl.ds(nbase, rows_per_sub)],
                        out_hbm.at[cid, pl.ds(nbase, rows_per_sub)])

    return scatter_k


def _table_body(h_ref, pos4_ref, w1r_ref, w1c_ref, tr_ref, tc_ref):
    h = h_ref[...]
    p4 = pos4_ref[...]
    pad = jnp.zeros((h.shape[0], DTBL - 68), jnp.float32)
    hr = jnp.dot(h, w1r_ref[...], preferred_element_type=jnp.float32)
    hc = jnp.dot(h, w1c_ref[...], preferred_element_type=jnp.float32)
    tr_ref[...] = jnp.concatenate([hr, p4, pad], axis=-1)
    tc_ref[...] = jnp.concatenate([hc, p4, pad], axis=-1)


def _edge_body(n_real, blk, gr_ref, gc_ref, w1d_ref, be1_ref, we2_ref, be2_ref,
               wc1_ref, bc1_ref, wc2_ref, bc2_ref, out_ref):
    gr = gr_ref[...]
    gc = gc_ref[...]
    diff = gr[:, 64:68] - gc[:, 64:68]
    dist2 = jnp.sum(diff * diff, axis=-1, keepdims=True)
    pre = gr[:, :64] + gc[:, :64] + dist2 * w1d_ref[...] + be1_ref[...]
    a1 = _silu(pre)
    m = _silu(jnp.dot(a1, we2_ref[...], preferred_element_type=jnp.float32)
              + be2_ref[...])
    t = _silu(jnp.dot(m, wc1_ref[...], preferred_element_type=jnp.float32)
              + bc1_ref[...])
    c = jnp.sum(t * wc2_ref[...], axis=-1, keepdims=True) + bc2_ref[0, 0]
    eid = pl.program_id(0) * blk + lax.broadcasted_iota(jnp.int32, (blk, 1), 0)
    live = jnp.where(eid < n_real, 1.0, 0.0).astype(jnp.float32)
    m = m * live
    fc = diff * (c * live)
    pad = jnp.zeros((blk, DTBL - 68), jnp.float32)
    out_ref[...] = jnp.concatenate([m, fc, pad], axis=-1)


def _node_body(blk, h_ref, g_ref, pos4_ref, vel4_ref, p0_ref, p1_ref,
               wn1h_ref, wn1a_ref, bn1_ref, wn2_ref, bn2_ref,
               wqh_ref, wqa_ref, bq_ref, nw1r_ref, nw1c_ref,
               h_out, g_out, pos4_out, vel4_out, tr_out, tc_out, qs_out):
    agg = p0_ref[0] + p1_ref[0]
    aggm = agg[:, :64]
    f4 = agg[:, 64:68]
    h = h_ref[...]
    pre1 = (jnp.dot(h, wn1h_ref[...], preferred_element_type=jnp.float32)
            + jnp.dot(aggm, wn1a_ref[...], preferred_element_type=jnp.float32)
            + bn1_ref[...])
    big_g = (jnp.dot(_silu(pre1), wn2_ref[...],
                     preferred_element_type=jnp.float32) + bn2_ref[...])
    q = (jnp.sum(h * wqh_ref[...], axis=-1, keepdims=True)
         + jnp.sum(aggm * wqa_ref[...], axis=-1, keepdims=True)
         + bq_ref[0, 0])
    q = jnp.tanh(q)
    vel4 = jnp.exp(q) * vel4_ref[...] + f4 * DT
    g_new = g_ref[...] + big_g * DT
    pos4 = jnp.mod(pos4_ref[...] + vel4 * DT, BOX)
    h_new = h + g_new * DT
    h_out[...] = h_new
    g_out[...] = g_new
    pos4_out[...] = pos4
    vel4_out[...] = vel4
    pad = jnp.zeros((blk, DTBL - 68), jnp.float32)
    nr = jnp.dot(h_new, nw1r_ref[...], preferred_element_type=jnp.float32)
    nc2 = jnp.dot(h_new, nw1c_ref[...], preferred_element_type=jnp.float32)
    tr_out[...] = jnp.concatenate([nr, pos4, pad], axis=-1)
    tc_out[...] = jnp.concatenate([nc2, pos4, pad], axis=-1)

    @pl.when(pl.program_id(0) == 0)
    def _():
        qs_out[...] = jnp.zeros((1, 1), jnp.float32)
    qs_out[...] += jnp.sum(q).reshape(1, 1)


def _full_spec(shape):
    return pl.BlockSpec(shape, lambda i: tuple(0 for _ in shape))


def kernel(h, g, pos, vel, edge_index, params):
    n = h.shape[0]
    e = edge_index.shape[1]
    e_pad = ((e + NW * CHUNK - 1) // (NW * CHUNK)) * (NW * CHUNK)

    row = edge_index[0]
    col = edge_index[1]
    pad_e = e_pad - e
    row2d = jnp.concatenate([row, jnp.zeros((pad_e,), jnp.int32)]).reshape(
        e_pad // SUB, SUB)
    col2d = jnp.concatenate([col, jnp.zeros((pad_e,), jnp.int32)]).reshape(
        e_pad // SUB, SUB)
    pos4 = jnp.concatenate([pos, jnp.zeros((n, 1), jnp.float32)], axis=-1)
    vel4 = jnp.concatenate([vel, jnp.zeros((n, 1), jnp.float32)], axis=-1)
    zeros_tbl = jnp.zeros((n, DTBL), jnp.float32)

    gather_k = _make_gather(e_pad)
    scatter_k = _make_scatter(e_pad, n)

    bn = 2000
    n_blocks = n // bn

    def table_call(hh, pp4, w1r, w1c):
        return pl.pallas_call(
            _table_body,
            grid=(n_blocks,),
            in_specs=[
                pl.BlockSpec((bn, 128), lambda i: (i, 0)),
                pl.BlockSpec((bn, 4), lambda i: (i, 0)),
                _full_spec((128, 64)),
                _full_spec((128, 64)),
            ],
            out_specs=[pl.BlockSpec((bn, DTBL), lambda i: (i, 0))] * 2,
            out_shape=[jax.ShapeDtypeStruct((n, DTBL), jnp.float32)] * 2,
        )(hh, pp4, w1r, w1c)

    be = 4096
    e_blocks = e_pad // be

    def edge_call(gr, gc, p):
        w1d = p['We1'][256:257, :]
        be1 = p['be1'].reshape(1, 64)
        be2 = p['be2'].reshape(1, 64)
        bc1 = p['bc1'].reshape(1, 64)
        wc2 = p['Wc2'].reshape(1, 64)
        bc2 = p['bc2'].reshape(1, 1)
        return pl.pallas_call(
            functools.partial(_edge_body, e, be),
            grid=(e_blocks,),
            in_specs=[
                pl.BlockSpec((be, DTBL), lambda i: (i, 0)),
                pl.BlockSpec((be, DTBL), lambda i: (i, 0)),
                _full_spec((1, 64)), _full_spec((1, 64)),
                _full_spec((64, 64)), _full_spec((1, 64)),
                _full_spec((64, 64)), _full_spec((1, 64)),
                _full_spec((1, 64)), _full_spec((1, 1)),
            ],
            out_specs=pl.BlockSpec((be, DTBL), lambda i: (i, 0)),
            out_shape=jax.ShapeDtypeStruct((e_pad, DTBL), jnp.float32),
        )(gr, gc, w1d, be1, p['We2'], be2, p['Wc1'], bc1, wc2, bc2)

    def node_call(hh, gg, pp4, vv4, parts, p, p_next):
        wn1h = p['Wn1'][:128, :]
        wn1a = p['Wn1'][128:, :]
        wqh = p['Wq'][:128, 0].reshape(1, 128)
        wqa = p['Wq'][128:, 0].reshape(1, 64)
        return pl.pallas_call(
            functools.partial(_node_body, bn),
            grid=(n_blocks,),
            in_specs=[
                pl.BlockSpec((bn, 128), lambda i: (i, 0)),
                pl.BlockSpec((bn, 128), lambda i: (i, 0)),
                pl.BlockSpec((bn, 4), lambda i: (i, 0)),
                pl.BlockSpec((bn, 4), lambda i: (i, 0)),
                pl.BlockSpec((1, bn, DTBL), lambda i: (0, i, 0)),
                pl.BlockSpec((1, bn, DTBL), lambda i: (1, i, 0)),
                _full_spec((128, 64)), _full_spec((64, 64)),
                _full_spec((1, 64)),
                _full_spec((64, 128)), _full_spec((1, 128)),
                _full_spec((1, 128)), _full_spec((1, 64)),
                _full_spec((1, 1)),
                _full_spec((128, 64)), _full_spec((128, 64)),
            ],
            out_specs=[
                pl.BlockSpec((bn, 128), lambda i: (i, 0)),
                pl.BlockSpec((bn, 128), lambda i: (i, 0)),
                pl.BlockSpec((bn, 4), lambda i: (i, 0)),
                pl.BlockSpec((bn, 4), lambda i: (i, 0)),
                pl.BlockSpec((bn, DTBL), lambda i: (i, 0)),
                pl.BlockSpec((bn, DTBL), lambda i: (i, 0)),
                pl.BlockSpec((1, 1), lambda i: (0, 0)),
            ],
            out_shape=[
                jax.ShapeDtypeStruct((n, 128), jnp.float32),
                jax.ShapeDtypeStruct((n, 128), jnp.float32),
                jax.ShapeDtypeStruct((n, 4), jnp.float32),
                jax.ShapeDtypeStruct((n, 4), jnp.float32),
                jax.ShapeDtypeStruct((n, DTBL), jnp.float32),
                jax.ShapeDtypeStruct((n, DTBL), jnp.float32),
                jax.ShapeDtypeStruct((1, 1), jnp.float32),
            ],
        )(hh, gg, pp4, vv4, parts, parts, wn1h, wn1a, p['bn1'].reshape(1, 64),
          p['Wn2'], p['bn2'].reshape(1, 128), wqh, wqa,
          p['bq'].reshape(1, 1), p_next['We1'][:128, :],
          p_next['We1'][128:256, :])

    p0 = params[0]
    tr, tc = table_call(h, pos4, p0['We1'][:128, :], p0['We1'][128:256, :])
    ldj = jnp.zeros((), jnp.float32)
    for it, p in enumerate(params):
        gr, gc = gather_k(row2d, col2d, tr, tc)
        vals = edge_call(gr, gc, p)
        parts = scatter_k(row2d, vals, zeros_tbl)
        p_next = params[it + 1] if it + 1 < len(params) else params[0]
        h, g, pos4, vel4, tr, tc, qs = node_call(
            h, g, pos4, vel4, parts, p, p_next)
        ldj = ldj + qs[0, 0]

    return h, g, pos4[:, :3], vel4[:, :3], ldj


# speedup vs baseline: 3.8767x; 1.0972x over previous
import functools

import jax
import jax.numpy as jnp
from jax import lax
from jax.experimental import pallas as pl
from jax.experimental.pallas import tpu as pltpu
from jax.experimental.pallas import tpu_sc as plsc

DT = 0.01
BOX = 10.0

NC = 2
NS = 16
NW = NC * NS
DTBL = 128
SUB = 128
CHUNK = 512
NSUB = CHUNK // SUB


def _silu(x):
    return x * jax.nn.sigmoid(x)


def _bf(x):
    return x.astype(jnp.bfloat16).astype(jnp.float32)


GRP = 1024
CPG = GRP // SUB


def _make_gather(e_pad):
    per_tile = e_pad // NW
    n_groups = per_tile // GRP
    mesh = plsc.VectorSubcoreMesh(
        core_axis_name="c", subcore_axis_name="s", num_cores=NC, num_subcores=NS)

    @functools.partial(
        pl.kernel, mesh=mesh,
        out_type=(jax.ShapeDtypeStruct((e_pad, DTBL), jnp.float32),
                  jax.ShapeDtypeStruct((e_pad, DTBL), jnp.float32)),
        scratch_types=[
            pltpu.VMEM((2, CPG, SUB), jnp.int32),
            pltpu.VMEM((2, CPG, SUB), jnp.int32),
            pltpu.VMEM((2, SUB, DTBL), jnp.float32),
            pltpu.VMEM((2, SUB, DTBL), jnp.float32),
            [pltpu.SemaphoreType.DMA] * 2,
            [pltpu.SemaphoreType.DMA] * 2,
            [pltpu.SemaphoreType.DMA] * 2,
        ],
    )
    def gather_k(row_hbm, col_hbm, tr_hbm, tc_hbm, gr_hbm, gc_hbm,
                 idx_r, idx_c, grbuf, gcbuf, semg, semo, semi):
        wid = lax.axis_index("s") * NC + lax.axis_index("c")
        g0 = wid * n_groups
        e0 = wid * per_tile

        def fetch_idx(g, gb):
            pltpu.async_copy(row_hbm.at[g0 + g], idx_r.at[gb], semi[gb])
            pltpu.async_copy(col_hbm.at[g0 + g], idx_c.at[gb], semi[gb])

        def wait_idx(gb):
            pltpu.make_async_copy(row_hbm.at[0], idx_r.at[gb], semi[gb]).wait()
            pltpu.make_async_copy(col_hbm.at[0], idx_c.at[gb], semi[gb]).wait()

        def issue_gathers(gb, j, b):
            pltpu.async_copy(tr_hbm.at[idx_r.at[gb, j]], grbuf.at[b], semg[b])
            pltpu.async_copy(tc_hbm.at[idx_c.at[gb, j]], gcbuf.at[b], semg[b])

        def wait_gathers(gb, j, b):
            pltpu.make_async_copy(tr_hbm.at[idx_r.at[gb, j]], grbuf.at[b],
                                  semg[b]).wait()
            pltpu.make_async_copy(tc_hbm.at[idx_c.at[gb, j]], gcbuf.at[b],
                                  semg[b]).wait()

        def issue_outs(g, j, b):
            ebase = e0 + g * GRP + j * SUB
            pltpu.async_copy(grbuf.at[b], gr_hbm.at[pl.ds(ebase, SUB)], semo[b])
            pltpu.async_copy(gcbuf.at[b], gc_hbm.at[pl.ds(ebase, SUB)], semo[b])

        def wait_outs(b):
            pltpu.make_async_copy(grbuf.at[b], gr_hbm.at[pl.ds(0, SUB)],
                                  semo[b]).wait()
            pltpu.make_async_copy(gcbuf.at[b], gc_hbm.at[pl.ds(0, SUB)],
                                  semo[b]).wait()

        fetch_idx(0, 0)
        wait_idx(0)

        def body(i, carry):
            for gb in (0, 1):
                g = 2 * i + gb
                @pl.when(g + 1 < n_groups)
                def _():
                    fetch_idx(g + 1, 1 - gb)
                issue_gathers(gb, 0, 0)
                issue_gathers(gb, 1, 1)
                for j in range(CPG):
                    b = j % 2
                    wait_gathers(gb, j, b)
                    issue_outs(g, j, b)
                    wait_outs(b)
                    if j + 2 < CPG:
                        issue_gathers(gb, j + 2, b)
                @pl.when(g + 1 < n_groups)
                def _():
                    wait_idx(1 - gb)
            return carry

        lax.fori_loop(0, n_groups // 2, body, 0)

    return gather_k


def _make_scatter(e_pad, n_pad):
    per_tile = e_pad // NW
    n_groups = per_tile // GRP
    vchunk = 128
    vsub = vchunk // SUB
    rows_per_sub = n_pad // NS
    mesh = plsc.VectorSubcoreMesh(
        core_axis_name="c", subcore_axis_name="s", num_cores=NC, num_subcores=NS)

    @functools.partial(
        pl.kernel, mesh=mesh,
        out_type=jax.ShapeDtypeStruct((NC, n_pad, DTBL), jnp.float32),
        scratch_types=[
            pltpu.VMEM_SHARED((n_pad, DTBL), jnp.float32),
            pltpu.VMEM((CPG, SUB), jnp.int32),
            pltpu.VMEM((vchunk, DTBL), jnp.float32),
        ],
    )
    def scatter_k(row_hbm, vals_hbm, zeros_hbm, out_hbm, acc, idx_r, vbuf):
        cid = lax.axis_index("c")
        sid = lax.axis_index("s")
        wid = sid * NC + cid
        nbase = sid * rows_per_sub
        pltpu.sync_copy(zeros_hbm.at[pl.ds(nbase, rows_per_sub)],
                        acc.at[pl.ds(nbase, rows_per_sub)])
        plsc.subcore_barrier()

        def group(g, carry):
            pltpu.sync_copy(row_hbm.at[wid * n_groups + g], idx_r)
            for v in range(GRP // vchunk):
                ebase = wid * per_tile + g * GRP + v * vchunk
                pltpu.sync_copy(vals_hbm.at[pl.ds(ebase, vchunk)], vbuf)
                for j in range(vsub):
                    pltpu.sync_copy(vbuf.at[pl.ds(j * SUB, SUB)],
                                    acc.at[idx_r.at[v * vsub + j]], add=True)
            return carry

        lax.fori_loop(0, n_groups, group, 0)
        plsc.subcore_barrier()
        pltpu.sync_copy(acc.at[pl.ds(nbase, rows_per_sub)],
                        out_hbm.at[cid, pl.ds(nbase, rows_per_sub)])

    return scatter_k


def _table_body(h_ref, pos4_ref, w1r_ref, w1c_ref, tr_ref, tc_ref):
    h = h_ref[...]
    p4 = pos4_ref[...]
    pad = jnp.zeros((h.shape[0], DTBL - 68), jnp.float32)
    hr = jnp.dot(h, w1r_ref[...], preferred_element_type=jnp.float32)
    hc = jnp.dot(h, w1c_ref[...], preferred_element_type=jnp.float32)
    tr_ref[...] = jnp.concatenate([hr, p4, pad], axis=-1)
    tc_ref[...] = jnp.concatenate([hc, p4, pad], axis=-1)


def _edge_body(n_real, blk, gr_ref, gc_ref, w1d_ref, be1_ref, we2_ref, be2_ref,
               wc1_ref, bc1_ref, wc2_ref, bc2_ref, out_ref):
    gr = gr_ref[...]
    gc = gc_ref[...]
    diff = gr[:, 64:68] - gc[:, 64:68]
    dist2 = jnp.sum(diff * diff, axis=-1, keepdims=True)
    pre = (gr[:, :64] + gc[:, :64] + _bf(dist2) * _bf(w1d_ref[...])
           + be1_ref[...])
    a1 = _silu(pre)
    m = _silu(jnp.dot(a1, we2_ref[...], preferred_element_type=jnp.float32)
              + be2_ref[...])
    t = _silu(jnp.dot(m, wc1_ref[...], preferred_element_type=jnp.float32)
              + bc1_ref[...])
    c = (jnp.sum(_bf(t) * _bf(wc2_ref[...]), axis=-1, keepdims=True)
         + bc2_ref[0, 0])
    eid = pl.program_id(0) * blk + lax.broadcasted_iota(jnp.int32, (blk, 1), 0)
    live = jnp.where(eid < n_real, 1.0, 0.0).astype(jnp.float32)
    m = m * live
    fc = diff * (c * live)
    pad = jnp.zeros((blk, DTBL - 68), jnp.float32)
    out_ref[...] = jnp.concatenate([m, fc, pad], axis=-1)


def _node_body(blk, h_ref, g_ref, pos4_ref, vel4_ref, p0_ref, p1_ref,
               wn1h_ref, wn1a_ref, bn1_ref, wn2_ref, bn2_ref,
               wqh_ref, wqa_ref, bq_ref, nw1r_ref, nw1c_ref,
               h_out, g_out, pos4_out, vel4_out, tr_out, tc_out, qs_out):
    agg = p0_ref[0] + p1_ref[0]
    aggm = agg[:, :64]
    f4 = agg[:, 64:68]
    h = h_ref[...]
    pre1 = (jnp.dot(h, wn1h_ref[...], preferred_element_type=jnp.float32)
            + jnp.dot(aggm, wn1a_ref[...], preferred_element_type=jnp.float32)
            + bn1_ref[...])
    big_g = (jnp.dot(_silu(pre1), wn2_ref[...],
                     preferred_element_type=jnp.float32) + bn2_ref[...])
    q = (jnp.sum(_bf(h) * _bf(wqh_ref[...]), axis=-1, keepdims=True)
         + jnp.sum(_bf(aggm) * _bf(wqa_ref[...]), axis=-1, keepdims=True)
         + bq_ref[0, 0])
    q = jnp.tanh(q)
    vel4 = jnp.exp(q) * vel4_ref[...] + f4 * DT
    g_new = g_ref[...] + big_g * DT
    pos4 = jnp.mod(pos4_ref[...] + vel4 * DT, BOX)
    h_new = h + g_new * DT
    h_out[...] = h_new
    g_out[...] = g_new
    pos4_out[...] = pos4
    vel4_out[...] = vel4
    pad = jnp.zeros((blk, DTBL - 68), jnp.float32)
    nr = jnp.dot(h_new, nw1r_ref[...], preferred_element_type=jnp.float32)
    nc2 = jnp.dot(h_new, nw1c_ref[...], preferred_element_type=jnp.float32)
    tr_out[...] = jnp.concatenate([nr, pos4, pad], axis=-1)
    tc_out[...] = jnp.concatenate([nc2, pos4, pad], axis=-1)

    @pl.when(pl.program_id(0) == 0)
    def _():
        qs_out[...] = jnp.zeros((1, 1), jnp.float32)
    qs_out[...] += jnp.sum(q).reshape(1, 1)


def _full_spec(shape):
    return pl.BlockSpec(shape, lambda i: tuple(0 for _ in shape))


def kernel(h, g, pos, vel, edge_index, params):
    n = h.shape[0]
    e = edge_index.shape[1]
    e_pad = ((e + NW * CHUNK - 1) // (NW * CHUNK)) * (NW * CHUNK)

    row = edge_index[0]
    col = edge_index[1]
    pad_e = e_pad - e
    n_pad = ((n + NS * 8 - 1) // (NS * 8)) * (NS * 8)
    row3d = jnp.concatenate([row, jnp.zeros((pad_e,), jnp.int32)]).reshape(
        e_pad // GRP, CPG, SUB)
    col3d = jnp.concatenate([col, jnp.zeros((pad_e,), jnp.int32)]).reshape(
        e_pad // GRP, CPG, SUB)
    pos4 = jnp.concatenate([pos, jnp.zeros((n, 1), jnp.float32)], axis=-1)
    vel4 = jnp.concatenate([vel, jnp.zeros((n, 1), jnp.float32)], axis=-1)
    zeros_tbl = jnp.zeros((n_pad, DTBL), jnp.float32)

    gather_k = _make_gather(e_pad)
    scatter_k = _make_scatter(e_pad, n_pad)

    bn = 2000
    n_blocks = n // bn

    def table_call(hh, pp4, w1r, w1c):
        return pl.pallas_call(
            _table_body,
            grid=(n_blocks,),
            in_specs=[
                pl.BlockSpec((bn, 128), lambda i: (i, 0)),
                pl.BlockSpec((bn, 4), lambda i: (i, 0)),
                _full_spec((128, 64)),
                _full_spec((128, 64)),
            ],
            out_specs=[pl.BlockSpec((bn, DTBL), lambda i: (i, 0))] * 2,
            out_shape=[jax.ShapeDtypeStruct((n, DTBL), jnp.float32)] * 2,
        )(hh, pp4, w1r, w1c)

    be = 4096
    e_blocks = e_pad // be

    def edge_call(gr, gc, p):
        w1d = p['We1'][256:257, :]
        be1 = p['be1'].reshape(1, 64)
        be2 = p['be2'].reshape(1, 64)
        bc1 = p['bc1'].reshape(1, 64)
        wc2 = p['Wc2'].reshape(1, 64)
        bc2 = p['bc2'].reshape(1, 1)
        return pl.pallas_call(
            functools.partial(_edge_body, e, be),
            grid=(e_blocks,),
            in_specs=[
                pl.BlockSpec((be, DTBL), lambda i: (i, 0)),
                pl.BlockSpec((be, DTBL), lambda i: (i, 0)),
                _full_spec((1, 64)), _full_spec((1, 64)),
                _full_spec((64, 64)), _full_spec((1, 64)),
                _full_spec((64, 64)), _full_spec((1, 64)),
                _full_spec((1, 64)), _full_spec((1, 1)),
            ],
            out_specs=pl.BlockSpec((be, DTBL), lambda i: (i, 0)),
            out_shape=jax.ShapeDtypeStruct((e_pad, DTBL), jnp.float32),
        )(gr, gc, w1d, be1, p['We2'], be2, p['Wc1'], bc1, wc2, bc2)

    def node_call(hh, gg, pp4, vv4, parts, p, p_next):
        wn1h = p['Wn1'][:128, :]
        wn1a = p['Wn1'][128:, :]
        wqh = p['Wq'][:128, 0].reshape(1, 128)
        wqa = p['Wq'][128:, 0].reshape(1, 64)
        return pl.pallas_call(
            functools.partial(_node_body, bn),
            grid=(n_blocks,),
            in_specs=[
                pl.BlockSpec((bn, 128), lambda i: (i, 0)),
                pl.BlockSpec((bn, 128), lambda i: (i, 0)),
                pl.BlockSpec((bn, 4), lambda i: (i, 0)),
                pl.BlockSpec((bn, 4), lambda i: (i, 0)),
                pl.BlockSpec((1, bn, DTBL), lambda i: (0, i, 0)),
                pl.BlockSpec((1, bn, DTBL), lambda i: (1, i, 0)),
                _full_spec((128, 64)), _full_spec((64, 64)),
                _full_spec((1, 64)),
                _full_spec((64, 128)), _full_spec((1, 128)),
                _full_spec((1, 128)), _full_spec((1, 64)),
                _full_spec((1, 1)),
                _full_spec((128, 64)), _full_spec((128, 64)),
            ],
            out_specs=[
                pl.BlockSpec((bn, 128), lambda i: (i, 0)),
                pl.BlockSpec((bn, 128), lambda i: (i, 0)),
                pl.BlockSpec((bn, 4), lambda i: (i, 0)),
                pl.BlockSpec((bn, 4), lambda i: (i, 0)),
                pl.BlockSpec((bn, DTBL), lambda i: (i, 0)),
                pl.BlockSpec((bn, DTBL), lambda i: (i, 0)),
                pl.BlockSpec((1, 1), lambda i: (0, 0)),
            ],
            out_shape=[
                jax.ShapeDtypeStruct((n, 128), jnp.float32),
                jax.ShapeDtypeStruct((n, 128), jnp.float32),
                jax.ShapeDtypeStruct((n, 4), jnp.float32),
                jax.ShapeDtypeStruct((n, 4), jnp.float32),
                jax.ShapeDtypeStruct((n, DTBL), jnp.float32),
                jax.ShapeDtypeStruct((n, DTBL), jnp.float32),
                jax.ShapeDtypeStruct((1, 1), jnp.float32),
            ],
        )(hh, gg, pp4, vv4, parts, parts, wn1h, wn1a, p['bn1'].reshape(1, 64),
          p['Wn2'], p['bn2'].reshape(1, 128), wqh, wqa,
          p['bq'].reshape(1, 1), p_next['We1'][:128, :],
          p_next['We1'][128:256, :])

    p0 = params[0]
    tr, tc = table_call(h, pos4, p0['We1'][:128, :], p0['We1'][128:256, :])
    ldj = jnp.zeros((), jnp.float32)
    for it, p in enumerate(params):
        gr, gc = gather_k(row3d, col3d, tr, tc)
        vals = edge_call(gr, gc, p)
        parts = scatter_k(row3d, vals, zeros_tbl)
        p_next = params[it + 1] if it + 1 < len(params) else params[0]
        h, g, pos4, vel4, tr, tc, qs = node_call(
            h, g, pos4, vel4, parts, p, p_next)
        ldj = ldj + qs[0, 0]

    return h, g, pos4[:, :3], vel4[:, :3], ldj
